# baseline (speedup 1.0000x reference)
import functools

import jax
import jax.numpy as jnp
from jax import lax
from jax.experimental import pallas as pl
from jax.experimental.pallas import tpu as pltpu
from jax.experimental.pallas import tpu_sc as plsc

_N = 10000
_E = 320000
_IN = 226
_H = 64
_G = 16
_NP = 10240

_NC = 2
_NS = 16
_NW = _NC * _NS
_EW = _E // _NW
_CH = 80
_GB = 5
_NCH = _EW // _CH

_RB = 5000


def _sc_mesh():
    return plsc.VectorSubcoreMesh(core_axis_name="c", subcore_axis_name="s")


@functools.partial(
    pl.kernel,
    out_type=jax.ShapeDtypeStruct((_NC, _NP), jnp.float32),
    mesh=_sc_mesh(),
    scratch_types=[
        pltpu.VMEM((_NCH, _CH), jnp.int32),
        pltpu.VMEM((_CH,), jnp.float32),
        pltpu.VMEM_SHARED((_NP,), jnp.float32),
    ],
    compiler_params=pltpu.CompilerParams(use_tc_tiling_on_sc=False),
)
def _deg_kernel(dst_hbm, zeros_hbm, out_hbm, dstv, onesv, acc):
    c = lax.axis_index("c")
    s = lax.axis_index("s")
    wid = s * _NC + c
    ones16 = jnp.full((16,), 1.0, dtype=jnp.float32)
    for i in range(_CH // 16):
        onesv[pl.ds(i * 16, 16)] = ones16
    rows = _NP // _NS
    pltpu.sync_copy(zeros_hbm.at[pl.ds(s * rows, rows)],
                    acc.at[pl.ds(s * rows, rows)])
    pltpu.sync_copy(dst_hbm.at[wid], dstv)
    plsc.subcore_barrier()

    def body(j, carry):
        pltpu.sync_copy(onesv, acc.at[dstv.at[j]], add=True)
        return carry

    lax.fori_loop(0, _NCH, body, 0)
    plsc.subcore_barrier()
    pltpu.sync_copy(acc.at[pl.ds(s * rows, rows)],
                    out_hbm.at[c, pl.ds(s * rows, rows)])


@functools.partial(
    pl.kernel,
    out_type=jax.ShapeDtypeStruct((_NC, _NP, _H), jnp.float32),
    mesh=_sc_mesh(),
    scratch_types=[
        pltpu.VMEM((_NCH, _CH), jnp.int32),
        pltpu.VMEM((_NCH, _CH), jnp.int32),
        pltpu.VMEM((_CH, _H), jnp.float32),
        pltpu.VMEM((2 * _GB, _CH, _H), jnp.float32),
        pltpu.VMEM_SHARED((_NP, _H), jnp.float32),
        pltpu.SemaphoreType.DMA,
        pltpu.SemaphoreType.DMA,
    ],
    compiler_params=pltpu.CompilerParams(use_tc_tiling_on_sc=False),
)
def _prop_kernel(y_hbm, src_hbm, dst_hbm, zeros_hbm, out_hbm,
                 srcv, dstv, zbuf, rowsv, acc, gsem, ssem):
    c = lax.axis_index("c")
    s = lax.axis_index("s")
    wid = s * _NC + c
    nrows = _NP // _NS
    pltpu.async_copy(zeros_hbm, zbuf, gsem)
    pltpu.async_copy(src_hbm.at[wid], srcv, gsem)
    pltpu.async_copy(dst_hbm.at[wid], dstv, gsem)
    pltpu.make_async_copy(zeros_hbm, zbuf, gsem).wait()
    for k in range(nrows // _CH):
        pltpu.async_copy(zbuf, acc.at[pl.ds(s * nrows + k * _CH, _CH)], ssem)
    pltpu.make_async_copy(src_hbm.at[wid], srcv, gsem).wait()
    pltpu.make_async_copy(dst_hbm.at[wid], dstv, gsem).wait()
    for k in range(nrows // _CH):
        pltpu.make_async_copy(zbuf, acc.at[pl.ds(s * nrows + k * _CH, _CH)],
                              ssem).wait()
    plsc.subcore_barrier()

    ng = _NCH // _GB
    for b in range(_GB):
        pltpu.async_copy(y_hbm.at[srcv.at[b]], rowsv.at[b], gsem)

    def gbody(g, carry):
        p = lax.rem(g, 2)
        for b in range(_GB):
            j = g * _GB + b
            buf = p * _GB + b
            pltpu.make_async_copy(y_hbm.at[srcv.at[j]], rowsv.at[buf],
                                  gsem).wait()
            pltpu.async_copy(rowsv.at[buf], acc.at[dstv.at[j]], ssem,
                             add=True)

        @pl.when(g < ng - 1)
        def _next_gathers():
            for b in range(_GB):
                j2 = (g + 1) * _GB + b
                pltpu.async_copy(y_hbm.at[srcv.at[j2]],
                                 rowsv.at[(1 - p) * _GB + b], gsem)

        for b in range(_GB):
            pltpu.make_async_copy(rowsv.at[p * _GB + b],
                                  acc.at[dstv.at[g * _GB + b]], ssem).wait()
        return carry

    lax.fori_loop(0, ng, gbody, 0)
    plsc.subcore_barrier()
    pltpu.sync_copy(acc.at[pl.ds(s * nrows, nrows)],
                    out_hbm.at[c, pl.ds(s * nrows, nrows)])


def _tc1_body(x_ref, w_ref, d0_ref, d1_ref, y_ref):
    dinv = lax.rsqrt(1.0 + d0_ref[...] + d1_ref[...])
    xw = jnp.dot(x_ref[...], w_ref[...], preferred_element_type=jnp.float32)
    y_ref[...] = xw * dinv


def _tcl_body(y_ref, pa_ref, pb_ref, d0_ref, d1_ref, b_ref, w_ref, yn_ref):
    dinv = lax.rsqrt(1.0 + d0_ref[...] + d1_ref[...])
    h = dinv * (y_ref[...] + pa_ref[0] + pb_ref[0]) + b_ref[...]
    h = jnp.maximum(h, 0.0)
    yn_ref[...] = jnp.dot(h, w_ref[...], preferred_element_type=jnp.float32) * dinv


def _final_body(y_ref, pa_ref, pb_ref, d0_ref, d1_ref, b_ref, batch_ref,
                wlin_ref, blin_ref, out_ref, sums_ref, cnts_ref):
    i = pl.program_id(0)

    @pl.when(i == 0)
    def _init():
        sums_ref[...] = jnp.zeros_like(sums_ref)
        cnts_ref[...] = jnp.zeros_like(cnts_ref)

    dinv = lax.rsqrt(1.0 + d0_ref[...] + d1_ref[...])
    h = dinv * (y_ref[...] + pa_ref[0] + pb_ref[0]) + b_ref[...]
    h = jnp.maximum(h, 0.0)
    g = lax.broadcasted_iota(jnp.int32, (1, _G), 1)
    mask = (batch_ref[...] == g).astype(jnp.float32)
    sums_ref[...] += lax.dot_general(
        mask, h, (((0,), (0,)), ((), ())), preferred_element_type=jnp.float32)
    ones = jnp.ones((_RB, 1), jnp.float32)
    cnts_ref[...] += lax.dot_general(
        mask, ones, (((0,), (0,)), ((), ())), preferred_element_type=jnp.float32)

    @pl.when(i == pl.num_programs(0) - 1)
    def _fin():
        pooled = sums_ref[...] / jnp.maximum(cnts_ref[...], 1.0)
        out_ref[...] = (jnp.dot(pooled, wlin_ref[...],
                                preferred_element_type=jnp.float32)
                        + blin_ref[...])


def _row_spec(i_dim):
    return pl.BlockSpec((_RB, i_dim), lambda i: (i, 0))


def _part_spec(core):
    return pl.BlockSpec((1, _RB, _H), lambda i, c=core: (c, i, 0))


def _const_spec(shape):
    nd = len(shape)
    return pl.BlockSpec(shape, lambda i: (0,) * nd)


def kernel(x, edge_index, batch, W1, b1, W2, b2, W3, b3, Wlin, blin):
    src2d = edge_index[0].reshape(_NW, _NCH, _CH)
    dst2d = edge_index[1].reshape(_NW, _NCH, _CH)
    zeros1 = jnp.zeros((_NP,), jnp.float32)
    zeros2 = jnp.zeros((_CH, _H), jnp.float32)

    degp = _deg_kernel(dst2d, zeros1)
    deg0 = degp[0, :_N, None]
    deg1 = degp[1, :_N, None]

    grid = (_N // _RB,)

    y1 = pl.pallas_call(
        _tc1_body,
        grid=grid,
        in_specs=[
            _row_spec(_IN),
            _const_spec((_IN, _H)),
            _row_spec(1),
            _row_spec(1),
        ],
        out_specs=_row_spec(_H),
        out_shape=jax.ShapeDtypeStruct((_N, _H), jnp.float32),
    )(x, W1, deg0, deg1)

    def layer(y, b, w):
        p = _prop_kernel(y, src2d, dst2d, zeros2)
        return pl.pallas_call(
            _tcl_body,
            grid=grid,
            in_specs=[
                _row_spec(_H), _part_spec(0), _part_spec(1),
                _row_spec(1), _row_spec(1),
                _const_spec((1, _H)), _const_spec((_H, _H)),
            ],
            out_specs=_row_spec(_H),
            out_shape=jax.ShapeDtypeStruct((_N, _H), jnp.float32),
        )(y, p, p, deg0, deg1, b.reshape(1, _H), w)

    y2 = layer(y1, b1, W2)
    y3 = layer(y2, b2, W3)

    p3 = _prop_kernel(y3, src2d, dst2d, zeros2)
    out = pl.pallas_call(
        _final_body,
        grid=grid,
        in_specs=[
            _row_spec(_H), _part_spec(0), _part_spec(1),
            _row_spec(1), _row_spec(1),
            _const_spec((1, _H)),
            _row_spec(1),
            _const_spec((_H, 1)), _const_spec((1, 1)),
        ],
        out_specs=_const_spec((_G, 1)),
        out_shape=jax.ShapeDtypeStruct((_G, 1), jnp.float32),
        scratch_shapes=[
            pltpu.VMEM((_G, _H), jnp.float32),
            pltpu.VMEM((_G, 1), jnp.float32),
        ],
    )(y3, p3, p3, deg0, deg1, b3.reshape(1, _H),
      batch[:, None], Wlin, blin.reshape(1, 1))
    return out


# speedup vs baseline: 1.0210x; 1.0210x over previous
import functools

import jax
import jax.numpy as jnp
from jax import lax
from jax.experimental import pallas as pl
from jax.experimental.pallas import tpu as pltpu
from jax.experimental.pallas import tpu_sc as plsc

_N = 10000
_E = 320000
_IN = 226
_H = 64
_G = 16
_NP = 10240

_NC = 2
_NS = 16
_NW = _NC * _NS
_EW = _E // _NW
_CH = 80
_GB = 5
_NCH = _EW // _CH

_RB = 2000


def _sc_mesh():
    return plsc.VectorSubcoreMesh(core_axis_name="c", subcore_axis_name="s")


@functools.partial(
    pl.kernel,
    out_type=jax.ShapeDtypeStruct((_NC, _NP), jnp.float32),
    mesh=_sc_mesh(),
    scratch_types=[
        pltpu.VMEM((_NCH, _CH), jnp.int32),
        pltpu.VMEM((_CH,), jnp.float32),
        pltpu.VMEM_SHARED((_NP,), jnp.float32),
        pltpu.SemaphoreType.DMA,
        pltpu.SemaphoreType.DMA,
    ],
    compiler_params=pltpu.CompilerParams(use_tc_tiling_on_sc=False),
)
def _deg_kernel(dst_hbm, zeros_hbm, out_hbm, dstv, onesv, acc, csem, ssem):
    c = lax.axis_index("c")
    s = lax.axis_index("s")
    wid = s * _NC + c
    ones16 = jnp.full((16,), 1.0, dtype=jnp.float32)
    for i in range(_CH // 16):
        onesv[pl.ds(i * 16, 16)] = ones16
    rows = _NP // _NS
    pltpu.async_copy(zeros_hbm.at[pl.ds(s * rows, rows)],
                     acc.at[pl.ds(s * rows, rows)], csem)
    pltpu.async_copy(dst_hbm.at[wid], dstv, csem)
    pltpu.make_async_copy(zeros_hbm.at[pl.ds(s * rows, rows)],
                          acc.at[pl.ds(s * rows, rows)], csem).wait()
    pltpu.make_async_copy(dst_hbm.at[wid], dstv, csem).wait()
    plsc.subcore_barrier()

    def body(j, carry):
        pltpu.async_copy(onesv, acc.at[dstv.at[j]], ssem, add=True)
        return carry

    lax.fori_loop(0, _NCH, body, 0)

    def bwait(j, carry):
        pltpu.make_async_copy(onesv, acc.at[dstv.at[j]], ssem).wait()
        return carry

    lax.fori_loop(0, _NCH, bwait, 0)
    plsc.subcore_barrier()
    pltpu.sync_copy(acc.at[pl.ds(s * rows, rows)],
                    out_hbm.at[c, pl.ds(s * rows, rows)])


@functools.partial(
    pl.kernel,
    out_type=jax.ShapeDtypeStruct((_NC, _NP, _H), jnp.float32),
    mesh=_sc_mesh(),
    scratch_types=[
        pltpu.VMEM((_NCH, _CH), jnp.int32),
        pltpu.VMEM((_NCH, _CH), jnp.int32),
        pltpu.VMEM((_CH, _H), jnp.float32),
        pltpu.VMEM((2 * _GB, _CH, _H), jnp.float32),
        pltpu.VMEM_SHARED((_NP, _H), jnp.float32),
        pltpu.SemaphoreType.DMA,
        pltpu.SemaphoreType.DMA,
    ],
    compiler_params=pltpu.CompilerParams(use_tc_tiling_on_sc=False),
)
def _prop_kernel(y_hbm, src_hbm, dst_hbm, zeros_hbm, out_hbm,
                 srcv, dstv, zbuf, rowsv, acc, gsem, ssem):
    c = lax.axis_index("c")
    s = lax.axis_index("s")
    wid = s * _NC + c
    nrows = _NP // _NS
    pltpu.async_copy(zeros_hbm, zbuf, gsem)
    pltpu.async_copy(src_hbm.at[wid], srcv, gsem)
    pltpu.async_copy(dst_hbm.at[wid], dstv, gsem)
    pltpu.make_async_copy(zeros_hbm, zbuf, gsem).wait()
    for k in range(nrows // _CH):
        pltpu.async_copy(zbuf, acc.at[pl.ds(s * nrows + k * _CH, _CH)], ssem)
    pltpu.make_async_copy(src_hbm.at[wid], srcv, gsem).wait()
    pltpu.make_async_copy(dst_hbm.at[wid], dstv, gsem).wait()
    for k in range(nrows // _CH):
        pltpu.make_async_copy(zbuf, acc.at[pl.ds(s * nrows + k * _CH, _CH)],
                              ssem).wait()
    plsc.subcore_barrier()

    ng = _NCH // _GB
    for b in range(_GB):
        pltpu.async_copy(y_hbm.at[srcv.at[b]], rowsv.at[b], gsem)

    def gbody(g, carry):
        p = lax.rem(g, 2)
        for b in range(_GB):
            j = g * _GB + b
            buf = p * _GB + b
            pltpu.make_async_copy(y_hbm.at[srcv.at[j]], rowsv.at[buf],
                                  gsem).wait()
            pltpu.async_copy(rowsv.at[buf], acc.at[dstv.at[j]], ssem,
                             add=True)

        @pl.when(g < ng - 1)
        def _next_gathers():
            for b in range(_GB):
                j2 = (g + 1) * _GB + b
                pltpu.async_copy(y_hbm.at[srcv.at[j2]],
                                 rowsv.at[(1 - p) * _GB + b], gsem)

        for b in range(_GB):
            pltpu.make_async_copy(rowsv.at[p * _GB + b],
                                  acc.at[dstv.at[g * _GB + b]], ssem).wait()
        return carry

    lax.fori_loop(0, ng, gbody, 0)
    plsc.subcore_barrier()
    pltpu.sync_copy(acc.at[pl.ds(s * nrows, nrows)],
                    out_hbm.at[c, pl.ds(s * nrows, nrows)])


def _tc1_body(x_ref, w_ref, d0_ref, d1_ref, y_ref):
    dinv = lax.rsqrt(1.0 + d0_ref[...] + d1_ref[...])
    xw = jnp.dot(x_ref[...], w_ref[...], preferred_element_type=jnp.float32)
    y_ref[...] = xw * dinv


def _tcl_body(y_ref, pa_ref, pb_ref, d0_ref, d1_ref, b_ref, w_ref, yn_ref):
    dinv = lax.rsqrt(1.0 + d0_ref[...] + d1_ref[...])
    h = dinv * (y_ref[...] + pa_ref[0] + pb_ref[0]) + b_ref[...]
    h = jnp.maximum(h, 0.0)
    yn_ref[...] = jnp.dot(h, w_ref[...], preferred_element_type=jnp.float32) * dinv


def _final_body(y_ref, pa_ref, pb_ref, d0_ref, d1_ref, b_ref, batch_ref,
                wlin_ref, blin_ref, out_ref, sums_ref, cnts_ref):
    i = pl.program_id(0)

    @pl.when(i == 0)
    def _init():
        sums_ref[...] = jnp.zeros_like(sums_ref)
        cnts_ref[...] = jnp.zeros_like(cnts_ref)

    dinv = lax.rsqrt(1.0 + d0_ref[...] + d1_ref[...])
    h = dinv * (y_ref[...] + pa_ref[0] + pb_ref[0]) + b_ref[...]
    h = jnp.maximum(h, 0.0)
    g = lax.broadcasted_iota(jnp.int32, (1, _G), 1)
    mask = (batch_ref[...] == g).astype(jnp.float32)
    sums_ref[...] += lax.dot_general(
        mask, h, (((0,), (0,)), ((), ())), preferred_element_type=jnp.float32)
    ones = jnp.ones((_RB, 1), jnp.float32)
    cnts_ref[...] += lax.dot_general(
        mask, ones, (((0,), (0,)), ((), ())), preferred_element_type=jnp.float32)

    @pl.when(i == pl.num_programs(0) - 1)
    def _fin():
        pooled = sums_ref[...] / jnp.maximum(cnts_ref[...], 1.0)
        out_ref[...] = (jnp.dot(pooled, wlin_ref[...],
                                preferred_element_type=jnp.float32)
                        + blin_ref[...])


def _row_spec(i_dim):
    return pl.BlockSpec((_RB, i_dim), lambda i: (i, 0))


def _part_spec(core):
    return pl.BlockSpec((1, _RB, _H), lambda i, c=core: (c, i, 0))


def _const_spec(shape):
    nd = len(shape)
    return pl.BlockSpec(shape, lambda i: (0,) * nd)


def kernel(x, edge_index, batch, W1, b1, W2, b2, W3, b3, Wlin, blin):
    src2d = edge_index[0].reshape(_NW, _NCH, _CH)
    dst2d = edge_index[1].reshape(_NW, _NCH, _CH)
    zeros1 = jnp.zeros((_NP,), jnp.float32)
    zeros2 = jnp.zeros((_CH, _H), jnp.float32)

    degp = _deg_kernel(dst2d, zeros1)
    deg0 = degp[0, :_N, None]
    deg1 = degp[1, :_N, None]

    grid = (_N // _RB,)

    y1 = pl.pallas_call(
        _tc1_body,
        grid=grid,
        in_specs=[
            _row_spec(_IN),
            _const_spec((_IN, _H)),
            _row_spec(1),
            _row_spec(1),
        ],
        out_specs=_row_spec(_H),
        out_shape=jax.ShapeDtypeStruct((_N, _H), jnp.float32),
    )(x, W1, deg0, deg1)

    def layer(y, b, w):
        p = _prop_kernel(y, src2d, dst2d, zeros2)
        return pl.pallas_call(
            _tcl_body,
            grid=grid,
            in_specs=[
                _row_spec(_H), _part_spec(0), _part_spec(1),
                _row_spec(1), _row_spec(1),
                _const_spec((1, _H)), _const_spec((_H, _H)),
            ],
            out_specs=_row_spec(_H),
            out_shape=jax.ShapeDtypeStruct((_N, _H), jnp.float32),
        )(y, p, p, deg0, deg1, b.reshape(1, _H), w)

    y2 = layer(y1, b1, W2)
    y3 = layer(y2, b2, W3)

    p3 = _prop_kernel(y3, src2d, dst2d, zeros2)
    out = pl.pallas_call(
        _final_body,
        grid=grid,
        in_specs=[
            _row_spec(_H), _part_spec(0), _part_spec(1),
            _row_spec(1), _row_spec(1),
            _const_spec((1, _H)),
            _row_spec(1),
            _const_spec((_H, 1)), _const_spec((1, 1)),
        ],
        out_specs=_const_spec((_G, 1)),
        out_shape=jax.ShapeDtypeStruct((_G, 1), jnp.float32),
        scratch_shapes=[
            pltpu.VMEM((_G, _H), jnp.float32),
            pltpu.VMEM((_G, 1), jnp.float32),
        ],
    )(y3, p3, p3, deg0, deg1, b3.reshape(1, _H),
      batch[:, None], Wlin, blin.reshape(1, 1))
    return out
